# baseline (speedup 1.0000x reference)
import jax
import jax.numpy as jnp
from jax.experimental import pallas as pl
from jax.experimental.pallas import tpu as pltpu

_HIDDEN = 4096
_E = 64
_K = 8
_BT = 512
_NB = 8192 // _BT


def _router_kernel(x_ref, w_ref, topw_ref, topi_ref, logits_ref, lbuf):
    i = pl.program_id(0)

    @pl.when(i < _NB)
    def _matmul():
        logits = jax.lax.dot_general(
            x_ref[...], w_ref[...],
            dimension_numbers=(((1,), (1,)), ((), ())),
            preferred_element_type=jnp.float32,
        )
        lbuf[i % 2] = logits
        logits_ref[...] = logits

    @pl.when(i > 0)
    def _route():
        logits = lbuf[(i - 1) % 2]
        _RC = 64
        iota = jax.lax.broadcasted_iota(
            jnp.int32, (_RC, _E), 1).astype(jnp.float32)
        for c in range(_BT // _RC):
            l = logits[c * _RC:(c + 1) * _RC, :]
            m = jnp.max(l, axis=1, keepdims=True)
            e = jnp.exp(l - m)
            s = jnp.sum(e, axis=1, keepdims=True)
            vals = e / s
            tops = []
            idxs = []
            total = jnp.zeros((_RC, 1), jnp.float32)
            for _ in range(_K):
                mv = jnp.max(vals, axis=1, keepdims=True)
                ix = jnp.min(jnp.where(vals == mv, iota, float(_E)),
                             axis=1, keepdims=True)
                tops.append(mv)
                idxs.append(ix)
                total = total + mv
                vals = jnp.where(iota == ix, -jnp.inf, vals)
            for j in range(_K):
                topw_ref[c * _RC:(c + 1) * _RC, j:j + 1] = tops[j] / total
                topi_ref[c * _RC:(c + 1) * _RC, j:j + 1] = (
                    idxs[j].astype(jnp.int32))


@jax.jit
def kernel(x, W_gate):
    tokens = x.shape[0]
    topw, topi, logits = pl.pallas_call(
        _router_kernel,
        grid=(_NB + 1,),
        in_specs=[
            pl.BlockSpec((_BT, _HIDDEN), lambda i: (jnp.minimum(i, _NB - 1), 0)),
            pl.BlockSpec((_E, _HIDDEN), lambda i: (0, 0)),
        ],
        out_specs=[
            pl.BlockSpec((_BT, _K), lambda i: (jnp.maximum(i - 1, 0), 0)),
            pl.BlockSpec((_BT, _K), lambda i: (jnp.maximum(i - 1, 0), 0)),
            pl.BlockSpec((_BT, _E), lambda i: (jnp.minimum(i, _NB - 1), 0)),
        ],
        out_shape=[
            jax.ShapeDtypeStruct((tokens, _K), jnp.float32),
            jax.ShapeDtypeStruct((tokens, _K), jnp.int32),
            jax.ShapeDtypeStruct((tokens, _E), jnp.float32),
        ],
        scratch_shapes=[pltpu.VMEM((2, _BT, _E), jnp.float32)],
    )(x, W_gate)
    return topw, topi, logits


# speedup vs baseline: 1.1368x; 1.1368x over previous
import jax
import jax.numpy as jnp
from jax.experimental import pallas as pl
from jax.experimental.pallas import tpu as pltpu

_HIDDEN = 4096
_E = 64
_K = 8
_BT = 512
_NB = 8192 // _BT


def _router_kernel(x_ref, w_ref, topw_ref, topi_ref, logits_ref, lbuf):
    i = pl.program_id(0)

    logits = jax.lax.dot_general(
        x_ref[...], w_ref[...],
        dimension_numbers=(((1,), (1,)), ((), ())),
        preferred_element_type=jnp.float32,
    )
    logits_ref[...] = logits

    prev = lbuf[(i - 1) % 2]
    _RC = 64
    iota = jax.lax.broadcasted_iota(
        jnp.int32, (_RC, _E), 1).astype(jnp.float32)
    for c in range(_BT // _RC):
        l = prev[c * _RC:(c + 1) * _RC, :]
        m = jnp.max(l, axis=1, keepdims=True)
        e = jnp.exp(l - m)
        s = jnp.sum(e, axis=1, keepdims=True)
        vals = e / s
        tops = []
        idxs = []
        total = jnp.zeros((_RC, 1), jnp.float32)
        for _ in range(_K):
            mv = jnp.max(vals, axis=1, keepdims=True)
            ix = jnp.min(jnp.where(vals == mv, iota, float(_E)),
                         axis=1, keepdims=True)
            tops.append(mv)
            idxs.append(ix)
            total = total + mv
            vals = jnp.where(iota == ix, -jnp.inf, vals)
        for j in range(_K):
            topw_ref[c * _RC:(c + 1) * _RC, j:j + 1] = tops[j] / total
            topi_ref[c * _RC:(c + 1) * _RC, j:j + 1] = (
                idxs[j].astype(jnp.int32))

    lbuf[i % 2] = logits


@jax.jit
def kernel(x, W_gate):
    tokens = x.shape[0]
    topw, topi, logits = pl.pallas_call(
        _router_kernel,
        grid=(_NB + 1,),
        in_specs=[
            pl.BlockSpec((_BT, _HIDDEN), lambda i: (jnp.minimum(i, _NB - 1), 0)),
            pl.BlockSpec((_E, _HIDDEN), lambda i: (0, 0)),
        ],
        out_specs=[
            pl.BlockSpec((_BT, _K), lambda i: (jnp.maximum(i - 1, 0), 0)),
            pl.BlockSpec((_BT, _K), lambda i: (jnp.maximum(i - 1, 0), 0)),
            pl.BlockSpec((_BT, _E), lambda i: (jnp.minimum(i, _NB - 1), 0)),
        ],
        out_shape=[
            jax.ShapeDtypeStruct((tokens, _K), jnp.float32),
            jax.ShapeDtypeStruct((tokens, _K), jnp.int32),
            jax.ShapeDtypeStruct((tokens, _E), jnp.float32),
        ],
        scratch_shapes=[pltpu.VMEM((2, _BT, _E), jnp.float32)],
    )(x, W_gate)
    return topw, topi, logits
